# baseline (speedup 1.0000x reference)
import functools

import jax
import jax.numpy as jnp
from jax import lax
from jax.experimental import pallas as pl
from jax.experimental.pallas import tpu as pltpu
from jax.experimental.pallas import tpu_sc as plsc

C_IN = 8
H = 16
W = 16
KH = 3
KW = 3
OC = 16
K = 2
HO = H - KH + 1
WO = W - KW + 1
S = HO * WO
N_RF = C_IN * KH * KW
B = 32
T = OC * S * N_RF

NC = 2
NS = 16
NW = 28
S_PER_W = S // NW

XLEN = B * C_IN * H * W
ROWS_OC = S_PER_W * N_RF
CI_OC = ROWS_OC * K
WT_W = 4 * OC * ROWS_OC
OUT_W = S_PER_W * B * OC

_mesh = plsc.VectorSubcoreMesh(core_axis_name="c", subcore_axis_name="s")


@functools.partial(
    pl.kernel,
    mesh=_mesh,
    compiler_params=pltpu.CompilerParams(needs_layout_passes=False),
    out_type=jax.ShapeDtypeStruct((S * B * OC,), jnp.float32),
    scratch_types=[
        pltpu.VMEM((XLEN,), jnp.float32),
        pltpu.VMEM((OC * CI_OC,), jnp.int32),
        pltpu.VMEM((WT_W,), jnp.float32),
        pltpu.VMEM((OUT_W,), jnp.float32),
        pltpu.SemaphoreType.DMA,
    ],
)
def _lutconv_sc(x_hbm, ci_hbm, wt_hbm, out_hbm, x_v, ci_v, wt_v, o_v, sem):
    wid = lax.axis_index("s") * NC + lax.axis_index("c")

    @pl.when(wid < NW)
    def _body():
        copies = [pltpu.async_copy(x_hbm, x_v, sem)]
        for oc in range(OC):
            copies.append(pltpu.async_copy(
                ci_hbm.at[pl.ds(oc * (S * N_RF * K) + wid * CI_OC, CI_OC)],
                ci_v.at[pl.ds(oc * CI_OC, CI_OC)], sem))
        for j in range(4):
            for oc in range(OC):
                copies.append(pltpu.async_copy(
                    wt_hbm.at[pl.ds(j * T + oc * (S * N_RF) + wid * ROWS_OC,
                                    ROWS_OC)],
                    wt_v.at[pl.ds((j * OC + oc) * ROWS_OC, ROWS_OC)], sem))
        for h in copies:
            h.wait()

        zero = jnp.zeros((OC,), jnp.float32)
        iota = lax.iota(jnp.int32, OC)
        ioc_ci = iota * CI_OC
        ioc_row = iota * ROWS_OC

        for si in range(S_PER_W):
            o_base0 = si * B * OC

            for b in range(B):
                o_v[pl.ds(o_base0 + b * OC, OC)] = zero

            def rf_body(rf, sw0, si=si, o_base0=o_base0):
                cib = ioc_ci + (si * (N_RF * K) + rf * K)
                rv = ioc_row + (si * N_RF + rf)
                ci0 = plsc.load_gather(ci_v, [cib])
                ci1 = plsc.load_gather(ci_v, [cib + 1])
                w0 = plsc.load_gather(wt_v, [rv])
                w1 = plsc.load_gather(wt_v, [rv + (OC * ROWS_OC)])
                w2 = plsc.load_gather(wt_v, [rv + 2 * (OC * ROWS_OC)])
                w3 = plsc.load_gather(wt_v, [rv + 3 * (OC * ROWS_OC)])
                bb = w2 - w0
                cc = w1 - w0
                aa = (w3 + w0) - (w1 + w2)
                for b in range(B):
                    off = b * (C_IN * H * W)
                    p0 = plsc.load_gather(x_v, [ci0 + off])
                    p1 = plsc.load_gather(x_v, [ci1 + off])
                    v = p0 * bb + (p1 * cc + (p0 * p1) * aa)
                    plsc.addupdate(o_v.at[pl.ds(o_base0 + b * OC, OC)], v)
                return sw0 + w0

            sw0 = lax.fori_loop(0, N_RF, rf_body, zero, unroll=2)
            for b in range(B):
                o_v[pl.ds(o_base0 + b * OC, OC)] = (
                    o_v[pl.ds(o_base0 + b * OC, OC)] + sw0)

        pltpu.sync_copy(o_v, out_hbm.at[pl.ds(wid * OUT_W, OUT_W)])


def kernel(x, input_mask, lut_weights):
    xf = x.reshape(-1)
    flat = (input_mask[:, 0] * (H * W) + input_mask[:, 1] * W
            + input_mask[:, 2]).astype(jnp.int32)
    wt_cols = lut_weights.T.reshape(-1)
    out = _lutconv_sc(xf, flat, wt_cols)
    out = out.reshape(S, B, OC)
    return out.transpose(1, 2, 0).reshape(B, OC, HO, WO)


# speedup vs baseline: 1.7588x; 1.7588x over previous
import functools

import jax
import jax.numpy as jnp
from jax import lax
from jax.experimental import pallas as pl
from jax.experimental.pallas import tpu as pltpu
from jax.experimental.pallas import tpu_sc as plsc

C_IN = 8
H = 16
W = 16
KH = 3
KW = 3
OC = 16
K = 2
HO = H - KH + 1
WO = W - KW + 1
S = HO * WO
N_RF = C_IN * KH * KW
B = 32
T = OC * S * N_RF

NC = 2
NS = 16
NW = 28
S_PER_W = S // NW

XLEN = B * C_IN * H * W
ROWS_OC = S_PER_W * N_RF
CI_OC = ROWS_OC * K
WT_W = 4 * OC * ROWS_OC
OUT_W = S_PER_W * B * OC

_mesh = plsc.VectorSubcoreMesh(core_axis_name="c", subcore_axis_name="s")


@functools.partial(
    pl.kernel,
    mesh=_mesh,
    compiler_params=pltpu.CompilerParams(needs_layout_passes=False),
    out_type=jax.ShapeDtypeStruct((S * B * OC,), jnp.float32),
    scratch_types=[
        pltpu.VMEM((XLEN,), jnp.float32),
        pltpu.VMEM((OC * CI_OC,), jnp.int32),
        pltpu.VMEM((WT_W,), jnp.float32),
        pltpu.VMEM((OUT_W,), jnp.float32),
        pltpu.SemaphoreType.DMA,
    ],
)
def _lutconv_sc(x_hbm, ci_hbm, wt_hbm, out_hbm, x_v, ci_v, wt_v, o_v, sem):
    wid = lax.axis_index("s") * NC + lax.axis_index("c")

    @pl.when(wid < NW)
    def _body():
        copies = [pltpu.async_copy(x_hbm, x_v, sem)]
        for oc in range(OC):
            copies.append(pltpu.async_copy(
                ci_hbm.at[pl.ds(oc * (S * N_RF * K) + wid * CI_OC, CI_OC)],
                ci_v.at[pl.ds(oc * CI_OC, CI_OC)], sem))
        for j in range(4):
            for oc in range(OC):
                copies.append(pltpu.async_copy(
                    wt_hbm.at[pl.ds(j * T + oc * (S * N_RF) + wid * ROWS_OC,
                                    ROWS_OC)],
                    wt_v.at[pl.ds((j * OC + oc) * ROWS_OC, ROWS_OC)], sem))
        for h in copies:
            h.wait()

        zero = jnp.zeros((OC,), jnp.float32)
        iota = lax.iota(jnp.int32, OC)
        ioc_ci = iota * CI_OC
        ioc_row = iota * ROWS_OC

        BG = 8
        for si in range(S_PER_W):
            o_base0 = si * B * OC
            sw0 = zero

            for bg in range(0, B, BG):
                first = bg == 0

                def rf_body(rf, carry, si=si, bg=bg, first=first):
                    cib = ioc_ci + (si * (N_RF * K) + rf * K)
                    rv = ioc_row + (si * N_RF + rf)
                    ci0 = plsc.load_gather(ci_v, [cib])
                    ci1 = plsc.load_gather(ci_v, [cib + 1])
                    w0 = plsc.load_gather(wt_v, [rv])
                    w1 = plsc.load_gather(wt_v, [rv + (OC * ROWS_OC)])
                    w2 = plsc.load_gather(wt_v, [rv + 2 * (OC * ROWS_OC)])
                    w3 = plsc.load_gather(wt_v, [rv + 3 * (OC * ROWS_OC)])
                    bb = w2 - w0
                    cc = w1 - w0
                    aa = (w3 + w0) - (w1 + w2)
                    out = []
                    for i in range(BG):
                        off = (bg + i) * (C_IN * H * W)
                        p0 = plsc.load_gather(x_v, [ci0 + off])
                        p1 = plsc.load_gather(x_v, [ci1 + off])
                        out.append(carry[i]
                                   + (p0 * bb + (p1 * cc + (p0 * p1) * aa)))
                    if first:
                        out.append(carry[BG] + w0)
                    return tuple(out)

                init = (zero,) * (BG + 1 if first else BG)
                accs = lax.fori_loop(0, N_RF, rf_body, init)
                if first:
                    sw0 = accs[BG]
                for i in range(BG):
                    o_v[pl.ds(o_base0 + (bg + i) * OC, OC)] = accs[i] + sw0

        pltpu.sync_copy(o_v, out_hbm.at[pl.ds(wid * OUT_W, OUT_W)])


def kernel(x, input_mask, lut_weights):
    xf = x.reshape(-1)
    flat = (input_mask[:, 0] * (H * W) + input_mask[:, 1] * W
            + input_mask[:, 2]).astype(jnp.int32)
    wt_cols = lut_weights.T.reshape(-1)
    out = _lutconv_sc(xf, flat, wt_cols)
    out = out.reshape(S, B, OC)
    return out.transpose(1, 2, 0).reshape(B, OC, HO, WO)


# speedup vs baseline: 1.8841x; 1.0712x over previous
import functools

import jax
import jax.numpy as jnp
from jax import lax
from jax.experimental import pallas as pl
from jax.experimental.pallas import tpu as pltpu
from jax.experimental.pallas import tpu_sc as plsc

C_IN = 8
H = 16
W = 16
KH = 3
KW = 3
OC = 16
K = 2
HO = H - KH + 1
WO = W - KW + 1
S = HO * WO
N_RF = C_IN * KH * KW
B = 32
T = OC * S * N_RF

NC = 2
NS = 16
NW = 28
S_PER_W = S // NW

XLEN = B * C_IN * H * W
ROWS_OC = S_PER_W * N_RF
CI_OC = ROWS_OC * K
WT_W = 4 * OC * ROWS_OC
OUT_W = S_PER_W * B * OC

_mesh = plsc.VectorSubcoreMesh(core_axis_name="c", subcore_axis_name="s")


@functools.partial(
    pl.kernel,
    mesh=_mesh,
    compiler_params=pltpu.CompilerParams(needs_layout_passes=False),
    out_type=jax.ShapeDtypeStruct((S * B * OC,), jnp.float32),
    scratch_types=[
        pltpu.VMEM((XLEN,), jnp.float32),
        pltpu.VMEM((OC * CI_OC,), jnp.int32),
        pltpu.VMEM((WT_W,), jnp.float32),
        pltpu.VMEM((OUT_W,), jnp.float32),
        pltpu.SemaphoreType.DMA,
    ],
)
def _lutconv_sc(x_hbm, ci_hbm, wt_hbm, out_hbm, x_v, ci_v, wt_v, o_v, sem):
    wid = lax.axis_index("s") * NC + lax.axis_index("c")

    @pl.when(wid < NW)
    def _body():
        copies = [pltpu.async_copy(x_hbm, x_v, sem)]
        for oc in range(OC):
            copies.append(pltpu.async_copy(
                ci_hbm.at[pl.ds(oc * (S * N_RF * K) + wid * CI_OC, CI_OC)],
                ci_v.at[pl.ds(oc * CI_OC, CI_OC)], sem))
        for j in range(4):
            for oc in range(OC):
                copies.append(pltpu.async_copy(
                    wt_hbm.at[pl.ds(j * T + oc * (S * N_RF) + wid * ROWS_OC,
                                    ROWS_OC)],
                    wt_v.at[pl.ds((j * OC + oc) * ROWS_OC, ROWS_OC)], sem))
        for h in copies:
            h.wait()

        zero = jnp.zeros((OC,), jnp.float32)
        iota = lax.iota(jnp.int32, OC)
        ioc_ci = iota * CI_OC
        ioc_row = iota * ROWS_OC

        BG = 16
        for si in range(S_PER_W):
            o_base0 = si * B * OC
            sw0 = zero

            for bg in range(0, B, BG):
                first = bg == 0

                def rf_body(rf, carry, si=si, bg=bg, first=first):
                    cib = ioc_ci + (si * (N_RF * K) + rf * K)
                    rv = ioc_row + (si * N_RF + rf)
                    ci0 = plsc.load_gather(ci_v, [cib])
                    ci1 = plsc.load_gather(ci_v, [cib + 1])
                    w0 = plsc.load_gather(wt_v, [rv])
                    w1 = plsc.load_gather(wt_v, [rv + (OC * ROWS_OC)])
                    w2 = plsc.load_gather(wt_v, [rv + 2 * (OC * ROWS_OC)])
                    w3 = plsc.load_gather(wt_v, [rv + 3 * (OC * ROWS_OC)])
                    bb = w2 - w0
                    cc = w1 - w0
                    aa = (w3 + w0) - (w1 + w2)
                    out = []
                    for i in range(BG):
                        off = (bg + i) * (C_IN * H * W)
                        p0 = plsc.load_gather(x_v, [ci0 + off])
                        p1 = plsc.load_gather(x_v, [ci1 + off])
                        out.append(carry[i]
                                   + (p0 * bb + (p1 * cc + (p0 * p1) * aa)))
                    if first:
                        out.append(carry[BG] + w0)
                    return tuple(out)

                init = (zero,) * (BG + 1 if first else BG)
                accs = lax.fori_loop(0, N_RF, rf_body, init)
                if first:
                    sw0 = accs[BG]
                for i in range(BG):
                    o_v[pl.ds(o_base0 + (bg + i) * OC, OC)] = accs[i] + sw0

        pltpu.sync_copy(o_v, out_hbm.at[pl.ds(wid * OUT_W, OUT_W)])


def kernel(x, input_mask, lut_weights):
    xf = x.reshape(-1)
    flat = (input_mask[:, 0] * (H * W) + input_mask[:, 1] * W
            + input_mask[:, 2]).astype(jnp.int32)
    wt_cols = lut_weights.T.reshape(-1)
    out = _lutconv_sc(xf, flat, wt_cols)
    out = out.reshape(S, B, OC)
    return out.transpose(1, 2, 0).reshape(B, OC, HO, WO)
